# baseline (speedup 1.0000x reference)
import functools

import jax
import jax.numpy as jnp
from jax import lax
from jax.experimental import pallas as pl
from jax.experimental.pallas import tpu as pltpu
from jax.experimental.pallas import tpu_sc as plsc

_B = 16384
_M = 32
_V = 1000000
_NC = 2
_NS = 16
_NW = _NC * _NS
_CH = 128
_NCH = _B // (_NW * _CH)
_TBL = 3


def _sc_gather(pidx, x0, x1, x2):
  mesh = plsc.VectorSubcoreMesh(core_axis_name="c", subcore_axis_name="s")
  nrows = _B // _CH

  @functools.partial(
      pl.kernel,
      out_type=jax.ShapeDtypeStruct((_TBL, nrows, _CH, 128), jnp.float32),
      mesh=mesh,
      compiler_params=pltpu.CompilerParams(use_tc_tiling_on_sc=True),
      scratch_types=[
          pltpu.VMEM((_TBL, _NCH, _CH), jnp.int32),
          pltpu.VMEM((_NCH, _CH, 128), jnp.float32),
          pltpu.SemaphoreType.DMA,
      ],
  )
  def k(idx_hbm, t0, t1, t2, out_hbm, idx_v, rows_v, sem):
    w = lax.axis_index("s") * _NC + lax.axis_index("c")
    base = w * _NCH
    pltpu.sync_copy(idx_hbm.at[:, pl.ds(base, _NCH)], idx_v)
    tabs = (t0, t1, t2)
    for t in range(_TBL):
      copies = []
      for ch in range(_NCH):
        copies.append(
            pltpu.async_copy(tabs[t].at[idx_v.at[t, ch]],
                             rows_v.at[ch], sem))
      for c in copies:
        c.wait()
      pltpu.sync_copy(rows_v, out_hbm.at[t, pl.ds(base, _NCH)])

  return k(pidx, x0, x1, x2)


_BLK = 1024


def _mlp_body(dx, r0, r1, r2, c0, c1, c2, wb0, bb0, wb1, bb1,
              w0a, w0b, w0c, w0d, b0, w1, b1, w2, b2, out):
  x = jnp.maximum(dx[...] @ wb0[...] + bb0[...], 0.0)
  x = jnp.maximum(x @ wb1[...] + bb1[...], 0.0)
  h = x @ w0a[...] + b0[...]
  for rr, cc, ww in ((r0, c0, w0b), (r1, c1, w0c), (r2, c2, w0d)):
    rv = rr[...]
    cv = cc[...]
    s = jnp.zeros((rv.shape[0], _M), jnp.float32)
    for slot in range(4):
      m = (cv == slot).astype(jnp.float32)
      s = s + m * rv[:, slot * _M:(slot + 1) * _M]
    h = h + s @ ww[...]
  h = jnp.maximum(h, 0.0)
  h = jnp.maximum(h @ w1[...] + b1[...], 0.0)
  out[...] = jax.nn.sigmoid(h @ w2[...] + b2[...])


def _tc_mlp(dense_x, r0, r1, r2, c0, c1, c2, W_b0, b_b0, W_b1, b_b1,
            W_t0, b_t0, W_t1, b_t1, W_t2, b_t2):
  w0a, w0b, w0c, w0d = W_t0[:8], W_t0[8:40], W_t0[40:72], W_t0[72:104]
  row = lambda blk: pl.BlockSpec((_BLK, blk.shape[1]), lambda i: (i, 0))
  rep = lambda a: pl.BlockSpec(a.shape, lambda i: (0,) * a.ndim)
  args = (dense_x, r0, r1, r2, c0, c1, c2, W_b0, b_b0.reshape(1, 8), W_b1,
          b_b1.reshape(1, 8), w0a, w0b, w0c, w0d, b_t0.reshape(1, 32),
          W_t1, b_t1.reshape(1, 16), W_t2, b_t2.reshape(1, 1))
  in_specs = [row(a) for a in args[:7]] + [rep(a) for a in args[7:]]
  return pl.pallas_call(
      _mlp_body,
      grid=(_B // _BLK,),
      in_specs=in_specs,
      out_specs=pl.BlockSpec((_BLK, 1), lambda i: (i, 0)),
      out_shape=jax.ShapeDtypeStruct((_B, 1), jnp.float32),
  )(*args)


def kernel(dense_x, sparse_indices, sparse_offsets, emb0, emb1, emb2,
           W_b0, b_b0, W_b1, b_b1, W_t0, b_t0, W_t1, b_t1, W_t2, b_t2):
  del sparse_offsets
  nrows = _B // _CH
  pidx = (sparse_indices >> 2).reshape(_TBL, nrows, _CH)
  x0 = emb0.reshape(_V // 4, 128)
  x1 = emb1.reshape(_V // 4, 128)
  x2 = emb2.reshape(_V // 4, 128)
  rows = _sc_gather(pidx, x0, x1, x2).reshape(_TBL, _B, 128)
  cm = (sparse_indices & 3).reshape(_TBL, _B, 1)
  return _tc_mlp(dense_x, rows[0], rows[1], rows[2],
                 cm[0], cm[1], cm[2],
                 W_b0, b_b0, W_b1, b_b1, W_t0, b_t0, W_t1, b_t1, W_t2, b_t2)


# speedup vs baseline: 1.0256x; 1.0256x over previous
import functools

import jax
import jax.numpy as jnp
from jax import lax
from jax.experimental import pallas as pl
from jax.experimental.pallas import tpu as pltpu
from jax.experimental.pallas import tpu_sc as plsc

_B = 16384
_M = 32
_NC = 2
_NS = 16
_NW = _NC * _NS
_CH = 128
_NCH = _B // (_NW * _CH)
_BPW = _NCH * _CH
_TBL = 3


def _sc_gather(idx3d, emb0, emb1, emb2):
  mesh = plsc.VectorSubcoreMesh(core_axis_name="c", subcore_axis_name="s")

  @functools.partial(
      pl.kernel,
      out_type=jax.ShapeDtypeStruct((_TBL, _B, _M), jnp.float32),
      mesh=mesh,
      compiler_params=pltpu.CompilerParams(use_tc_tiling_on_sc=False),
      scratch_types=[
          pltpu.VMEM((_TBL, _NCH, _CH), jnp.int32),
          pltpu.VMEM((_TBL, _BPW, _M), jnp.float32),
          pltpu.SemaphoreType.DMA,
          pltpu.SemaphoreType.DMA,
      ],
  )
  def k(idx_hbm, e0, e1, e2, out_hbm, idx_v, rows_v, gsem, osem):
    w = lax.axis_index("s") * _NC + lax.axis_index("c")
    base = w * _NCH
    pltpu.sync_copy(idx_hbm.at[:, pl.ds(base, _NCH)], idx_v)
    embs = (e0, e1, e2)
    copies = []
    for t in range(_TBL):
      for ch in range(_NCH):
        copies.append(
            pltpu.async_copy(embs[t].at[idx_v.at[t, ch]],
                             rows_v.at[t, pl.ds(ch * _CH, _CH)], gsem))
    outs = []
    for t in range(_TBL):
      for ch in range(_NCH):
        copies[t * _NCH + ch].wait()
      outs.append(
          pltpu.async_copy(rows_v.at[t],
                           out_hbm.at[t, pl.ds(w * _BPW, _BPW)], osem))
    for o in outs:
      o.wait()

  return k(idx3d, emb0, emb1, emb2)


_BLK = 1024


def _mlp_body(dx, s0, s1, s2, wb0, bb0, wb1, bb1,
              w0a, w0b, w0c, w0d, b0, w1, b1, w2, b2, out):
  x = jnp.maximum(dx[...] @ wb0[...] + bb0[...], 0.0)
  x = jnp.maximum(x @ wb1[...] + bb1[...], 0.0)
  h = (x @ w0a[...] + s0[...] @ w0b[...] + s1[...] @ w0c[...]
       + s2[...] @ w0d[...] + b0[...])
  h = jnp.maximum(h, 0.0)
  h = jnp.maximum(h @ w1[...] + b1[...], 0.0)
  out[...] = jax.nn.sigmoid(h @ w2[...] + b2[...])


def _tc_mlp(dense_x, s0, s1, s2, W_b0, b_b0, W_b1, b_b1,
            W_t0, b_t0, W_t1, b_t1, W_t2, b_t2):
  w0a, w0b, w0c, w0d = W_t0[:8], W_t0[8:40], W_t0[40:72], W_t0[72:104]
  row = lambda blk: pl.BlockSpec((_BLK, blk.shape[1]), lambda i: (i, 0))
  rep = lambda a: pl.BlockSpec(a.shape, lambda i: (0,) * a.ndim)
  args = (dense_x, s0, s1, s2, W_b0, b_b0.reshape(1, 8), W_b1,
          b_b1.reshape(1, 8), w0a, w0b, w0c, w0d, b_t0.reshape(1, 32),
          W_t1, b_t1.reshape(1, 16), W_t2, b_t2.reshape(1, 1))
  in_specs = [row(dense_x), row(s0), row(s1), row(s2)] + [
      rep(a) for a in args[4:]]
  return pl.pallas_call(
      _mlp_body,
      grid=(_B // _BLK,),
      in_specs=in_specs,
      out_specs=pl.BlockSpec((_BLK, 1), lambda i: (i, 0)),
      out_shape=jax.ShapeDtypeStruct((_B, 1), jnp.float32),
  )(*args)


def kernel(dense_x, sparse_indices, sparse_offsets, emb0, emb1, emb2,
           W_b0, b_b0, W_b1, b_b1, W_t0, b_t0, W_t1, b_t1, W_t2, b_t2):
  del sparse_offsets
  idx3d = sparse_indices.reshape(_TBL, _B // _CH, _CH)
  rows = _sc_gather(idx3d, emb0, emb1, emb2)
  return _tc_mlp(dense_x, rows[0], rows[1], rows[2],
                 W_b0, b_b0, W_b1, b_b1, W_t0, b_t0, W_t1, b_t1, W_t2, b_t2)
